# baseline (speedup 1.0000x reference)
import jax
import jax.numpy as jnp
from jax import lax
from jax.experimental import pallas as pl
from jax.experimental.pallas import tpu as pltpu


_ROWS, _COLS = 32768, 768
_CHUNK_ROWS = 2048
_NBUF = 4
_NCHUNK = _ROWS // _CHUNK_ROWS


def _double_kernel(x_hbm, o_hbm, in_buf, out_buf, in_sem, out_sem):
    def in_copy(i, b):
        return pltpu.make_async_copy(
            x_hbm.at[pl.ds(i * _CHUNK_ROWS, _CHUNK_ROWS), :], in_buf.at[b],
            in_sem.at[b])

    def out_copy(i, b):
        return pltpu.make_async_copy(
            out_buf.at[b], o_hbm.at[pl.ds(i * _CHUNK_ROWS, _CHUNK_ROWS), :],
            out_sem.at[b])

    for i in range(_NBUF):
        in_copy(i, i).start()

    def step(i, carry):
        b = lax.rem(i, _NBUF)
        in_copy(i, b).wait()

        @pl.when(i >= _NBUF)
        def _():
            out_copy(i - _NBUF, b).wait()

        out_buf[b] = in_buf[b] + in_buf[b]
        out_copy(i, b).start()

        @pl.when(i + _NBUF < _NCHUNK)
        def _():
            in_copy(i + _NBUF, b).start()

        return carry

    lax.fori_loop(0, _NCHUNK, step, 0)

    for k in range(_NBUF):
        i = _NCHUNK - _NBUF + k
        out_copy(i, i % _NBUF).wait()


def kernel(x, Wg, bg, W1, b1, W2, b2):
    B, T, C = x.shape
    x2 = x.reshape(B * T, C)
    out = pl.pallas_call(
        _double_kernel,
        in_specs=[pl.BlockSpec(memory_space=pl.ANY)],
        out_specs=pl.BlockSpec(memory_space=pl.ANY),
        out_shape=jax.ShapeDtypeStruct((B * T, C), x.dtype),
        scratch_shapes=[
            pltpu.VMEM((_NBUF, _CHUNK_ROWS, C), jnp.float32),
            pltpu.VMEM((_NBUF, _CHUNK_ROWS, C), jnp.float32),
            pltpu.SemaphoreType.DMA((_NBUF,)),
            pltpu.SemaphoreType.DMA((_NBUF,)),
        ],
    )(x2)
    return out.reshape(B, T, C)
